# Initial kernel scaffold; baseline (speedup 1.0000x reference)
import jax
import jax.numpy as jnp
from jax.experimental import pallas as pl


def kernel(atoms, pos, edge_index, batch, params):
    raise NotImplementedError("write your pallas kernel here")


# speedup vs baseline: 2.4739x; 2.4739x over previous
import functools

import jax
import jax.numpy as jnp
import numpy as np
from jax import lax
from jax.experimental import pallas as pl
from jax.experimental.pallas import tpu as pltpu
from jax.experimental.pallas import tpu_sc as plsc

N = 10000
E = 320000
D = 144
D0 = 16
NC, NS = 2, 16
NW = NC * NS
EPW = E // NW
CH = 80
NCH = EPW // CH
NPS = N // NS
BE = 2000
BN = 2000

_F32 = jnp.float32


def _silu(x):
    return x * jax.nn.sigmoid(x)


def _sh5(u):
    x = u[:, 0:1]
    y = u[:, 1:2]
    z = u[:, 2:3]
    s3 = np.float32(np.sqrt(3.0))
    return jnp.concatenate(
        [s3 * x * z, s3 * x * y, y * y - 0.5 * (x * x + z * z),
         s3 * y * z, (s3 / 2.0) * (z * z - x * x)], axis=1)


_SC_PARAMS = pltpu.CompilerParams(use_tc_tiling_on_sc=False)


def _sc_gather(ta, tb, row, col):
    mesh = plsc.VectorSubcoreMesh(core_axis_name="c", subcore_axis_name="s")

    @functools.partial(
        pl.kernel, mesh=mesh, compiler_params=_SC_PARAMS,
        out_type=[jax.ShapeDtypeStruct((E, D), _F32),
                  jax.ShapeDtypeStruct((E, D), _F32)],
        scratch_types=[pltpu.VMEM((CH,), jnp.int32),
                       pltpu.VMEM((CH,), jnp.int32),
                       pltpu.VMEM((CH, D), _F32),
                       pltpu.VMEM((CH, D), _F32),
                       pltpu.SemaphoreType.DMA,
                       pltpu.SemaphoreType.DMA],
    )
    def k(ta_hbm, tb_hbm, row_hbm, col_hbm, ars_hbm, bcs_hbm,
          ridx, cidx, abuf, bbuf, sem_a, sem_b):
        wid = lax.axis_index("s") * NC + lax.axis_index("c")
        base = wid * EPW

        @pl.loop(0, NCH)
        def _(i):
            off = base + i * CH
            pltpu.sync_copy(row_hbm.at[pl.ds(off, CH)], ridx)
            pltpu.sync_copy(col_hbm.at[pl.ds(off, CH)], cidx)
            cp_a = pltpu.async_copy(ta_hbm.at[ridx], abuf, sem_a)
            cp_b = pltpu.async_copy(tb_hbm.at[cidx], bbuf, sem_b)
            cp_a.wait()
            cp_b.wait()
            pltpu.sync_copy(abuf, ars_hbm.at[pl.ds(off, CH)])
            pltpu.sync_copy(bbuf, bcs_hbm.at[pl.ds(off, CH)])

    return k(ta, tb, row, col)


def _sc_scatter(eo, row, zed, d):
    mesh = plsc.VectorSubcoreMesh(core_axis_name="c", subcore_axis_name="s")

    @functools.partial(
        pl.kernel, mesh=mesh, compiler_params=_SC_PARAMS,
        out_type=jax.ShapeDtypeStruct((NC, N, d), _F32),
        scratch_types=[pltpu.VMEM((CH,), jnp.int32),
                       pltpu.VMEM((CH, d), _F32),
                       pltpu.VMEM_SHARED((N, d), _F32)],
    )
    def k(eo_hbm, row_hbm, zed_hbm, out_hbm, idxv, vbuf, acc):
        c = lax.axis_index("c")
        s = lax.axis_index("s")
        pltpu.sync_copy(zed_hbm, acc.at[pl.ds(s * NPS, NPS)])
        plsc.subcore_barrier()
        wid = s * NC + c
        base = wid * EPW

        @pl.loop(0, NCH)
        def _(i):
            off = base + i * CH
            pltpu.sync_copy(row_hbm.at[pl.ds(off, CH)], idxv)
            pltpu.sync_copy(eo_hbm.at[pl.ds(off, CH)], vbuf)
            pltpu.sync_copy(vbuf, acc.at[idxv], add=True)

        plsc.subcore_barrier()
        pltpu.sync_copy(acc.at[pl.ds(s * NPS, NPS)],
                        out_hbm.at[c, pl.ds(s * NPS, NPS)])

    return k(eo, row, zed)


def _tc_prep(atoms2d, pos, emb, w1a, w1b):
    grid = N // BN

    def body(at_ref, pos_ref, emb_ref, wa_ref, wb_ref,
             feat_ref, ta_ref, tb_ref, ps_ref):
        i = pl.program_id(0)
        onehot = (at_ref[...] == lax.broadcasted_iota(jnp.int32, (1, 16), 1)
                  ).astype(_F32)
        feat = jnp.dot(onehot, emb_ref[...], preferred_element_type=_F32)
        feat_ref[...] = feat
        p = pos_ref[...]
        pad = jnp.zeros((BN, 13), _F32)
        ta_ref[...] = jnp.concatenate(
            [jnp.dot(feat, wa_ref[...], preferred_element_type=_F32), p, pad],
            axis=1)
        tb_ref[...] = jnp.concatenate(
            [jnp.dot(feat, wb_ref[...], preferred_element_type=_F32), p, pad],
            axis=1)
        psum = jnp.sum(jnp.concatenate([p, jnp.zeros((BN, 125), _F32)], 1),
                       axis=0, keepdims=True)
        psum8 = jnp.broadcast_to(psum, (8, 128))

        @pl.when(i == 0)
        def _():
            ps_ref[...] = jnp.zeros((8, 128), _F32)

        ps_ref[...] += psum8

    return pl.pallas_call(
        body,
        grid=(grid,),
        in_specs=[pl.BlockSpec((BN, 1), lambda i: (i, 0)),
                  pl.BlockSpec((BN, 3), lambda i: (i, 0)),
                  pl.BlockSpec((16, 128), lambda i: (0, 0)),
                  pl.BlockSpec((128, 128), lambda i: (0, 0)),
                  pl.BlockSpec((128, 128), lambda i: (0, 0))],
        out_specs=[pl.BlockSpec((BN, 128), lambda i: (i, 0)),
                   pl.BlockSpec((BN, D), lambda i: (i, 0)),
                   pl.BlockSpec((BN, D), lambda i: (i, 0)),
                   pl.BlockSpec((8, 128), lambda i: (0, 0))],
        out_shape=[jax.ShapeDtypeStruct((N, 128), _F32),
                   jax.ShapeDtypeStruct((N, D), _F32),
                   jax.ShapeDtypeStruct((N, D), _F32),
                   jax.ShapeDtypeStruct((8, 128), _F32)],
    )(atoms2d, pos, emb, w1a, w1b)


def _tc_edge_init(ars, bcs, vec):
    grid = E // BE

    def body(a_ref, b_ref, vec_ref, eo_ref):
        a = a_ref[:, 0:128]
        b = b_ref[:, 0:128]
        pr = a_ref[:, 128:131]
        pc = b_ref[:, 128:131]
        d = pr - pc
        dist2 = jnp.sum(d * d, axis=1, keepdims=True)
        dist = jnp.sqrt(dist2)
        h = _silu(a + b + dist * vec_ref[0:1, :] + vec_ref[1:2, :])
        w2 = (jnp.sum(h * vec_ref[2:3, :], axis=1, keepdims=True)
              + vec_ref[3:4, 0:1])
        u = d / jnp.maximum(dist, 1e-12)
        eo_ref[...] = jnp.concatenate(
            [w2 * _sh5(u), jnp.ones((BE, 1), _F32), jnp.zeros((BE, 10), _F32)],
            axis=1)

    return pl.pallas_call(
        body,
        grid=(grid,),
        in_specs=[pl.BlockSpec((BE, D), lambda i: (i, 0)),
                  pl.BlockSpec((BE, D), lambda i: (i, 0)),
                  pl.BlockSpec((8, 128), lambda i: (0, 0))],
        out_specs=pl.BlockSpec((BE, D0), lambda i: (i, 0)),
        out_shape=jax.ShapeDtypeStruct((E, D0), _F32),
    )(ars, bcs, vec)


def _tc_edge_layer(ars, bcs, w2, wp1, ws1, vec):
    grid = E // BE

    def body(a_ref, b_ref, w2_ref, wp1_ref, ws1_ref, vec_ref, eo_ref):
        a = a_ref[:, 0:128]
        b = b_ref[:, 0:128]
        pr = a_ref[:, 128:131]
        pc = b_ref[:, 128:131]
        sr = a_ref[:, 131:136]
        sc = b_ref[:, 131:136]
        d = pr - pc
        dist2 = jnp.sum(d * d, axis=1, keepdims=True)
        ip = jnp.sum(sr * sc, axis=1, keepdims=True)
        h1 = _silu(a + b + dist2 * vec_ref[0:1, :] + ip * vec_ref[1:2, :]
                   + vec_ref[2:3, :])
        msg = _silu(jnp.dot(h1, w2_ref[...], preferred_element_type=_F32)
                    + vec_ref[3:4, :])
        t = _silu(jnp.dot(msg, wp1_ref[...], preferred_element_type=_F32)
                  + vec_ref[4:5, :])
        pval = (jnp.sum(t * vec_ref[5:6, :], axis=1, keepdims=True)
                + vec_ref[8:9, 0:1])
        sv = _silu(jnp.dot(msg, ws1_ref[...], preferred_element_type=_F32)
                   + vec_ref[6:7, :])
        wsh = (jnp.sum(sv * vec_ref[7:8, :], axis=1, keepdims=True)
               + vec_ref[8:9, 1:2])
        eo_ref[...] = jnp.concatenate(
            [msg, d * pval, (sr - sc) * wsh, jnp.ones((BE, 1), _F32),
             jnp.zeros((BE, 7), _F32)], axis=1)

    return pl.pallas_call(
        body,
        grid=(grid,),
        in_specs=[pl.BlockSpec((BE, D), lambda i: (i, 0)),
                  pl.BlockSpec((BE, D), lambda i: (i, 0)),
                  pl.BlockSpec((128, 128), lambda i: (0, 0)),
                  pl.BlockSpec((128, 128), lambda i: (0, 0)),
                  pl.BlockSpec((128, 128), lambda i: (0, 0)),
                  pl.BlockSpec((16, 128), lambda i: (0, 0))],
        out_specs=pl.BlockSpec((BE, D), lambda i: (i, 0)),
        out_shape=jax.ShapeDtypeStruct((E, D), _F32),
    )(ars, bcs, w2, wp1, ws1, vec)


def _tc_node0(p0a, p0b, possum, pos, feat, wc1, vec, w1a, w1b):
    grid = N // BN

    def body(pa_ref, pb_ref, ps_ref, pos_ref, feat_ref, wc1_ref, vec_ref,
             wa_ref, wb_ref, ta_ref, tb_ref, sh_ref):
        agg = pa_ref[...] + pb_ref[...]
        cnt = jnp.maximum(agg[:, 5:6], 1.0)
        mean_dsh = agg[:, 0:5] / cnt
        pos_com = ps_ref[0:1, 0:3] * np.float32(1.0 / N)
        p = pos_ref[...]
        v = p - pos_com
        nv = jnp.sqrt(jnp.sum(v * v, axis=1, keepdims=True))
        uv = v / jnp.maximum(nv, 1e-12)
        feat = feat_ref[...]
        hc = _silu(jnp.dot(feat, wc1_ref[...], preferred_element_type=_F32)
                   + vec_ref[0:1, :])
        w2c = (jnp.sum(hc * vec_ref[1:2, :], axis=1, keepdims=True)
               + vec_ref[2:3, 0:1])
        sh = mean_dsh + w2c * _sh5(uv)
        pad = jnp.zeros((BN, 8), _F32)
        ta_ref[...] = jnp.concatenate(
            [jnp.dot(feat, wa_ref[...], preferred_element_type=_F32), p, sh,
             pad], axis=1)
        tb_ref[...] = jnp.concatenate(
            [jnp.dot(feat, wb_ref[...], preferred_element_type=_F32), p, sh,
             pad], axis=1)
        sh_ref[...] = jnp.concatenate([sh, jnp.zeros((BN, 11), _F32)], axis=1)

    return pl.pallas_call(
        body,
        grid=(grid,),
        in_specs=[pl.BlockSpec((BN, D0), lambda i: (i, 0)),
                  pl.BlockSpec((BN, D0), lambda i: (i, 0)),
                  pl.BlockSpec((8, 128), lambda i: (0, 0)),
                  pl.BlockSpec((BN, 3), lambda i: (i, 0)),
                  pl.BlockSpec((BN, 128), lambda i: (i, 0)),
                  pl.BlockSpec((128, 128), lambda i: (0, 0)),
                  pl.BlockSpec((8, 128), lambda i: (0, 0)),
                  pl.BlockSpec((128, 128), lambda i: (0, 0)),
                  pl.BlockSpec((128, 128), lambda i: (0, 0))],
        out_specs=[pl.BlockSpec((BN, D), lambda i: (i, 0)),
                   pl.BlockSpec((BN, D), lambda i: (i, 0)),
                   pl.BlockSpec((BN, D0), lambda i: (i, 0))],
        out_shape=[jax.ShapeDtypeStruct((N, D), _F32),
                   jax.ShapeDtypeStruct((N, D), _F32),
                   jax.ShapeDtypeStruct((N, D0), _F32)],
    )(p0a, p0b, possum, pos, feat, wc1, vec, w1a, w1b)


def _tc_node_layer(pa, pb, ta_old, sh_old, feat, wn1a, wn1b, wn2, vec,
                   w1a, w1b):
    grid = N // BN

    def body(pa_ref, pb_ref, ta_old_ref, sh_old_ref, feat_ref,
             wn1a_ref, wn1b_ref, wn2_ref, vec_ref, wa_ref, wb_ref,
             ta_ref, tb_ref, sh_ref, feat_out_ref):
        agg = pa_ref[...] + pb_ref[...]
        cnt = jnp.maximum(agg[:, 136:137], 1.0)
        msg_agg = agg[:, 0:128] / cnt
        pos_new = ta_old_ref[:, 128:131] + agg[:, 128:131] / cnt
        sh_new = sh_old_ref[:, 0:5] + agg[:, 131:136] / cnt
        feat = feat_ref[...]
        hn = _silu(jnp.dot(feat, wn1a_ref[...], preferred_element_type=_F32)
                   + jnp.dot(msg_agg, wn1b_ref[...],
                             preferred_element_type=_F32)
                   + vec_ref[0:1, :])
        feat_new = (jnp.dot(hn, wn2_ref[...], preferred_element_type=_F32)
                    + vec_ref[1:2, :])
        feat_out_ref[...] = feat_new
        pad = jnp.zeros((BN, 8), _F32)
        ta_ref[...] = jnp.concatenate(
            [jnp.dot(feat_new, wa_ref[...], preferred_element_type=_F32),
             pos_new, sh_new, pad], axis=1)
        tb_ref[...] = jnp.concatenate(
            [jnp.dot(feat_new, wb_ref[...], preferred_element_type=_F32),
             pos_new, sh_new, pad], axis=1)
        sh_ref[...] = jnp.concatenate([sh_new, jnp.zeros((BN, 11), _F32)],
                                      axis=1)

    return pl.pallas_call(
        body,
        grid=(grid,),
        in_specs=[pl.BlockSpec((BN, D), lambda i: (i, 0)),
                  pl.BlockSpec((BN, D), lambda i: (i, 0)),
                  pl.BlockSpec((BN, D), lambda i: (i, 0)),
                  pl.BlockSpec((BN, D0), lambda i: (i, 0)),
                  pl.BlockSpec((BN, 128), lambda i: (i, 0)),
                  pl.BlockSpec((128, 128), lambda i: (0, 0)),
                  pl.BlockSpec((128, 128), lambda i: (0, 0)),
                  pl.BlockSpec((128, 128), lambda i: (0, 0)),
                  pl.BlockSpec((8, 128), lambda i: (0, 0)),
                  pl.BlockSpec((128, 128), lambda i: (0, 0)),
                  pl.BlockSpec((128, 128), lambda i: (0, 0))],
        out_specs=[pl.BlockSpec((BN, D), lambda i: (i, 0)),
                   pl.BlockSpec((BN, D), lambda i: (i, 0)),
                   pl.BlockSpec((BN, D0), lambda i: (i, 0)),
                   pl.BlockSpec((BN, 128), lambda i: (i, 0))],
        out_shape=[jax.ShapeDtypeStruct((N, D), _F32),
                   jax.ShapeDtypeStruct((N, D), _F32),
                   jax.ShapeDtypeStruct((N, D0), _F32),
                   jax.ShapeDtypeStruct((N, 128), _F32)],
    )(pa, pb, ta_old, sh_old, feat, wn1a, wn1b, wn2, vec, w1a, w1b)


def _tc_final(pa, pb, ta_old, sh_old, feat, wn1a, wn1b, wn2, vec):
    grid = N // BN

    def body(pa_ref, pb_ref, ta_old_ref, sh_old_ref, feat_ref,
             wn1a_ref, wn1b_ref, wn2_ref, vec_ref, out_ref):
        i = pl.program_id(0)
        agg = pa_ref[...] + pb_ref[...]
        cnt = jnp.maximum(agg[:, 136:137], 1.0)
        msg_agg = agg[:, 0:128] / cnt
        pos_new = ta_old_ref[:, 128:131] + agg[:, 128:131] / cnt
        sh_new = sh_old_ref[:, 0:5] + agg[:, 131:136] / cnt
        feat = feat_ref[...]
        hn = _silu(jnp.dot(feat, wn1a_ref[...], preferred_element_type=_F32)
                   + jnp.dot(msg_agg, wn1b_ref[...],
                             preferred_element_type=_F32)
                   + vec_ref[0:1, :])
        feat_new = (jnp.dot(hn, wn2_ref[...], preferred_element_type=_F32)
                    + vec_ref[1:2, :])
        total = (jnp.sum(feat_new * vec_ref[2:3, :])
                 + jnp.sum(pos_new * vec_ref[3:4, 0:3])
                 + jnp.sum(sh_new * vec_ref[3:4, 3:8]))

        @pl.when(i == 0)
        def _():
            out_ref[...] = jnp.broadcast_to(vec_ref[3:4, 8:9], (8, 128))

        out_ref[...] += total

    return pl.pallas_call(
        body,
        grid=(grid,),
        in_specs=[pl.BlockSpec((BN, D), lambda i: (i, 0)),
                  pl.BlockSpec((BN, D), lambda i: (i, 0)),
                  pl.BlockSpec((BN, D), lambda i: (i, 0)),
                  pl.BlockSpec((BN, D0), lambda i: (i, 0)),
                  pl.BlockSpec((BN, 128), lambda i: (i, 0)),
                  pl.BlockSpec((128, 128), lambda i: (0, 0)),
                  pl.BlockSpec((128, 128), lambda i: (0, 0)),
                  pl.BlockSpec((128, 128), lambda i: (0, 0)),
                  pl.BlockSpec((8, 128), lambda i: (0, 0))],
        out_specs=pl.BlockSpec((8, 128), lambda i: (0, 0)),
        out_shape=jax.ShapeDtypeStruct((8, 128), _F32),
    )(pa, pb, ta_old, sh_old, feat, wn1a, wn1b, wn2, vec)


def _row_pack(rows, nrows=8):
    out = []
    for r in rows:
        r = jnp.asarray(r, _F32).reshape(-1)
        out.append(jnp.pad(r, (0, 128 - r.shape[0])))
    while len(out) < nrows:
        out.append(jnp.zeros((128,), _F32))
    return jnp.stack(out)


def kernel(atoms, pos, edge_index, batch, params):
    del batch
    row = edge_index[0]
    col = edge_index[1]
    atoms2d = atoms.reshape(N, 1)
    zed16 = jnp.zeros((NPS, D0), _F32)
    zed144 = jnp.zeros((NPS, D), _F32)

    psh = params["sh_init"]["mlp_sh"]
    vec0 = _row_pack([psh["l1"]["w"][0], psh["l1"]["b"], psh["l2"]["w"][:, 2],
                      psh["l2"]["b"][2:3]])
    pcom = params["sh_init"]["mlp_sh_com"]
    vec_com = _row_pack([pcom["l1"]["b"], pcom["l2"]["w"][:, 2],
                         pcom["l2"]["b"][2:3]])
    emb16 = params["embedding"]

    lw = []
    for lp in params["layers"]:
        m = lp["mlp_msg"]
        p = lp["mlp_pos"]
        s = lp["mlp_sh"]
        nf = lp["mlp_node_feat"]
        lw.append(dict(
            w1a=m["l1"]["w"][0:128], w1b=m["l1"]["w"][128:256],
            w2=m["l2"]["w"],
            wp1=p["l1"]["w"], ws1=s["l1"]["w"],
            vec=_row_pack([m["l1"]["w"][256], m["l1"]["w"][259],
                           m["l1"]["b"], m["l2"]["b"],
                           p["l1"]["b"], p["l2"]["w"][:, 0],
                           s["l1"]["b"], s["l2"]["w"][:, 2],
                           jnp.stack([p["l2"]["b"][0], s["l2"]["b"][2]])],
                          nrows=16),
            wn1a=nf["l1"]["w"][0:128], wn1b=nf["l1"]["w"][128:256],
            wn2=nf["l2"]["w"],
            nvec=_row_pack([nf["l1"]["b"], nf["l2"]["b"]]),
        ))
    pw = params["pred"]["w"][:, 0]
    pred_tail = jnp.concatenate([pw[128:131], pw[135:140],
                                 params["pred"]["b"][0:1]])
    fvec = _row_pack([lw[1]["nvec"][0], lw[1]["nvec"][1], pw[0:128],
                      pred_tail])

    feat0, ta0, tb0, possum = _tc_prep(
        atoms2d, pos, emb16, psh["l1"]["w"][1:129], psh["l1"]["w"][129:257])
    ars0, bcs0 = _sc_gather(ta0, tb0, row, col)
    eo0 = _tc_edge_init(ars0, bcs0, vec0)
    p0 = _sc_scatter(eo0, row, zed16, D0)
    ta1, tb1, sh1 = _tc_node0(
        p0[0], p0[1], possum, pos, feat0, pcom["l1"]["w"], vec_com,
        lw[0]["w1a"], lw[0]["w1b"])

    ars1, bcs1 = _sc_gather(ta1, tb1, row, col)
    eo1 = _tc_edge_layer(ars1, bcs1, lw[0]["w2"], lw[0]["wp1"], lw[0]["ws1"],
                         lw[0]["vec"])
    p1 = _sc_scatter(eo1, row, zed144, D)
    ta2, tb2, sh2, feat1 = _tc_node_layer(
        p1[0], p1[1], ta1, sh1, feat0,
        lw[0]["wn1a"], lw[0]["wn1b"], lw[0]["wn2"], lw[0]["nvec"],
        lw[1]["w1a"], lw[1]["w1b"])

    ars2, bcs2 = _sc_gather(ta2, tb2, row, col)
    eo2 = _tc_edge_layer(ars2, bcs2, lw[1]["w2"], lw[1]["wp1"], lw[1]["ws1"],
                         lw[1]["vec"])
    p2 = _sc_scatter(eo2, row, zed144, D)
    out = _tc_final(p2[0], p2[1], ta2, sh2, feat1,
                    lw[1]["wn1a"], lw[1]["wn1b"], lw[1]["wn2"], fvec)
    return out[0:1, 0:1]
